# baseline (speedup 1.0000x reference)
import functools

import jax
import jax.numpy as jnp
from jax import lax
from jax.experimental import pallas as pl
from jax.experimental.pallas import tpu as pltpu
from jax.experimental.pallas import tpu_sc as plsc

_N = 10000
_E = 160000
_GRAPHS = 64
_NC = 2
_NS = 16
_NW = _NC * _NS
_BLK = 128
_PER_W = 5120
_E_PAD = _NW * _PER_W
_NBLK = _PER_W // _BLK
_NPAD = 10112
_RPT = _NPAD // _NS

_mesh = plsc.VectorSubcoreMesh(core_axis_name="c", subcore_axis_name="s")


def _f32(*shape):
  return jax.ShapeDtypeStruct(shape, jnp.float32)


_NBUF = 3


@functools.partial(
    pl.kernel,
    out_type=_f32(_E_PAD, 128),
    mesh=_mesh,
    scratch_types=[
        pltpu.VMEM((_NBLK, _BLK), jnp.int32),
        [pltpu.VMEM((_BLK, 128), jnp.float32) for _ in range(_NBUF)],
        pltpu.SemaphoreType.DMA,
        pltpu.SemaphoreType.DMA,
    ],
)
def _gather128(table_hbm, idx2d_hbm, out_hbm, idxv, rows, gsem, wsem):
  wid = lax.axis_index("s") * _NC + lax.axis_index("c")
  base = wid * _PER_W
  brow = wid * _NBLK
  pltpu.sync_copy(idx2d_hbm.at[pl.ds(brow, _NBLK)], idxv)

  def gather_dma(j, buf):
    return pltpu.make_async_copy(table_hbm.at[idxv.at[j]], buf, gsem)

  def write_dma(j, buf):
    return pltpu.make_async_copy(
        buf, out_hbm.at[pl.ds(base + j * _BLK, _BLK)], wsem)

  gather_dma(0, rows[0]).start()
  for j in range(_NBLK):
    b = rows[j % _NBUF]
    gather_dma(j, b).wait()
    write_dma(j, b).start()
    if j >= 2:
      write_dma(j - 2, rows[(j - 2) % _NBUF]).wait()
    if j + 1 < _NBLK:
      gather_dma(j + 1, rows[(j + 1) % _NBUF]).start()
  write_dma(_NBLK - 2, rows[(_NBLK - 2) % _NBUF]).wait()
  write_dma(_NBLK - 1, rows[(_NBLK - 1) % _NBUF]).wait()


@functools.partial(
    pl.kernel,
    out_type=_f32(2, _NPAD, 128),
    mesh=_mesh,
    scratch_types=[
        pltpu.VMEM((_NBLK, _BLK), jnp.int32),
        pltpu.VMEM((_BLK, 128), jnp.float32),
        pltpu.VMEM_SHARED((_NPAD, 128), jnp.float32),
        pltpu.SemaphoreType.DMA,
    ],
)
def _scatter_k(t_hbm, dst2d_hbm, out_hbm, idxv, tbuf, acc, sem):
  core = lax.axis_index("c")
  sid = lax.axis_index("s")
  wid = sid * _NC + core
  base = wid * _PER_W
  zero16 = jnp.zeros((16,), jnp.float32)

  def zrow(r, carry):
    for k in range(8):
      tbuf[r, pl.ds(k * 16, 16)] = zero16
    return carry

  lax.fori_loop(0, _BLK, zrow, 0)
  r0 = sid * _RPT
  brow = wid * _NBLK
  for z in range(5):
    zo = min(z * _BLK, _RPT - _BLK)
    pltpu.sync_copy(tbuf, acc.at[pl.ds(r0 + zo, _BLK)])
  pltpu.sync_copy(dst2d_hbm.at[pl.ds(brow, _NBLK)], idxv)
  plsc.subcore_barrier()

  def step(j, carry):
    off = base + j * _BLK
    pltpu.sync_copy(t_hbm.at[pl.ds(off, _BLK)], tbuf)
    pltpu.sync_copy(tbuf, acc.at[idxv.at[j]], add=True)
    return carry

  lax.fori_loop(0, _NBLK, step, 0)
  plsc.subcore_barrier()
  pltpu.sync_copy(acc.at[pl.ds(r0, _RPT)], out_hbm.at[core, pl.ds(r0, _RPT)])


_BE = 1024


@functools.lru_cache(maxsize=None)
def _dense1():
  hc = 7 * 128

  def body(ps_ref, pd_ref, xj_ref, win_ref, bin_ref, wout_ref,
           t_ref, rel_ref):
    rel = ps_ref[:, :16] - pd_ref[:, :16]
    rel_ref[...] = rel
    ss = jnp.dot(rel, win_ref[...], preferred_element_type=jnp.float32)
    ss = jnp.maximum(ss + bin_ref[...], 0.0)
    xj = xj_ref[...]
    acc = jnp.zeros((_BE, 64), jnp.float32)
    for h in range(7):
      m = ss[:, h * 128:(h + 1) * 128] * xj
      acc = acc + jnp.dot(m, wout_ref[h], preferred_element_type=jnp.float32)
    t_ref[...] = jnp.concatenate(
        [acc, jnp.zeros((_BE, 64), jnp.float32)], axis=1)

  return pl.pallas_call(
      body,
      grid=(_E_PAD // _BE,),
      in_specs=[
          pl.BlockSpec((_BE, 128), lambda i: (i, 0)),
          pl.BlockSpec((_BE, 128), lambda i: (i, 0)),
          pl.BlockSpec((_BE, 128), lambda i: (i, 0)),
          pl.BlockSpec((16, hc), lambda i: (0, 0)),
          pl.BlockSpec((1, hc), lambda i: (0, 0)),
          pl.BlockSpec((7, 128, 64), lambda i: (0, 0, 0)),
      ],
      out_specs=[
          pl.BlockSpec((_BE, 128), lambda i: (i, 0)),
          pl.BlockSpec((_BE, 16), lambda i: (i, 0)),
      ],
      out_shape=[_f32(_E_PAD, 128), _f32(_E_PAD, 16)],
      compiler_params=pltpu.CompilerParams(
          dimension_semantics=("arbitrary",)),
  )


@functools.lru_cache(maxsize=None)
def _dense23(h_cnt):
  hc = h_cnt * 64

  def body(rel_ref, xj_ref, win_ref, bin_ref, wout_ref, t_ref):
    ss = jnp.dot(rel_ref[...], win_ref[...],
                 preferred_element_type=jnp.float32)
    ss = jnp.maximum(ss + bin_ref[...], 0.0)
    xj = xj_ref[:, :64]
    acc = jnp.zeros((_BE, 64), jnp.float32)
    for h in range(h_cnt):
      m = ss[:, h * 64:(h + 1) * 64] * xj
      acc = acc + jnp.dot(m, wout_ref[h], preferred_element_type=jnp.float32)
    t_ref[...] = jnp.concatenate(
        [acc, jnp.zeros((_BE, 64), jnp.float32)], axis=1)

  return pl.pallas_call(
      body,
      grid=(_E_PAD // _BE,),
      in_specs=[
          pl.BlockSpec((_BE, 16), lambda i: (i, 0)),
          pl.BlockSpec((_BE, 128), lambda i: (i, 0)),
          pl.BlockSpec((16, hc), lambda i: (0, 0)),
          pl.BlockSpec((1, hc), lambda i: (0, 0)),
          pl.BlockSpec((h_cnt, 64, 64), lambda i: (0, 0, 0)),
      ],
      out_specs=pl.BlockSpec((_BE, 128), lambda i: (i, 0)),
      out_shape=_f32(_E_PAD, 128),
      compiler_params=pltpu.CompilerParams(
          dimension_semantics=("arbitrary",)),
  )


_RB = 2000


def _combine_body(p_ref, b_ref, h_ref):
  h_ref[...] = jnp.maximum(p_ref[0] + p_ref[1] + b_ref[...], 0.0)


_combine = pl.pallas_call(
    _combine_body,
    grid=(_N // _RB,),
    in_specs=[
        pl.BlockSpec((2, _RB, 128), lambda i: (0, i, 0)),
        pl.BlockSpec((1, 128), lambda i: (0, 0)),
    ],
    out_specs=pl.BlockSpec((_RB, 128), lambda i: (i, 0)),
    out_shape=_f32(_N, 128),
    compiler_params=pltpu.CompilerParams(dimension_semantics=("arbitrary",)),
)


def _pool_body(p_ref, b_ref, batch_ref, wfc_ref, bfc_ref,
               out_ref, acc_ref):
  i = pl.program_id(0)

  @pl.when(i == 0)
  def _():
    acc_ref[...] = jnp.zeros_like(acc_ref)

  h3 = jnp.maximum(p_ref[0] + p_ref[1] + b_ref[...], 0.0)
  hc = jnp.concatenate([h3, jnp.ones((_RB, 64), jnp.float32)], axis=1)
  ids = batch_ref[0]
  g = lax.broadcasted_iota(jnp.int32, (_GRAPHS, _RB), 0)
  m = (g == ids).astype(jnp.float32)
  acc_ref[...] += jnp.dot(m, hc, preferred_element_type=jnp.float32)

  @pl.when(i == pl.num_programs(0) - 1)
  def _():
    sums = acc_ref[:, :64]
    cnt = acc_ref[:, 128:129]
    pooled = sums / jnp.maximum(cnt, 1.0)
    logits = jnp.dot(pooled, wfc_ref[...],
                     preferred_element_type=jnp.float32) + bfc_ref[...]
    mx = jnp.max(logits, axis=1, keepdims=True)
    lse = jnp.log(jnp.sum(jnp.exp(logits - mx), axis=1, keepdims=True))
    out_ref[...] = logits - mx - lse


_pool = pl.pallas_call(
    _pool_body,
    grid=(_N // _RB,),
    in_specs=[
        pl.BlockSpec((2, _RB, 128), lambda i: (0, i, 0)),
        pl.BlockSpec((1, 128), lambda i: (0, 0)),
        pl.BlockSpec((1, 1, _RB), lambda i: (i, 0, 0)),
        pl.BlockSpec((64, 10), lambda i: (0, 0)),
        pl.BlockSpec((1, 10), lambda i: (0, 0)),
    ],
    out_specs=pl.BlockSpec((_GRAPHS, 10), lambda i: (0, 0)),
    out_shape=_f32(_GRAPHS, 10),
    scratch_shapes=[pltpu.VMEM((_GRAPHS, 192), jnp.float32)],
    compiler_params=pltpu.CompilerParams(dimension_semantics=("arbitrary",)),
)


def _permute_weights(win, bi, wout, h_cnt, c_in):
  winp = win.reshape(3, c_in, h_cnt).transpose(0, 2, 1).reshape(3, h_cnt * c_in)
  winp = jnp.concatenate(
      [winp, jnp.zeros((13, h_cnt * c_in), jnp.float32)], axis=0)
  bip = bi.reshape(c_in, h_cnt).transpose(1, 0).reshape(1, h_cnt * c_in)
  woutp = wout.reshape(c_in, h_cnt, 64).transpose(1, 0, 2)
  return winp, bip, woutp


def kernel(x, pos, edge_index, batch, Win1, bin1, Wout1, bout1, Win2, bin2,
           Wout2, bout2, Win3, bin3, Wout3, bout3, Wfc, bfc):
  src = jnp.concatenate(
      [edge_index[0], jnp.zeros((_E_PAD - _E,), jnp.int32)]).reshape(
          _E_PAD // _BLK, _BLK)
  dst = jnp.concatenate(
      [edge_index[1], jnp.full((_E_PAD - _E,), _N, jnp.int32)]).reshape(
          _E_PAD // _BLK, _BLK)
  ptab = jnp.concatenate([pos, jnp.zeros((_N, 125), jnp.float32)], axis=1)

  w1 = _permute_weights(Win1, bin1, Wout1, 7, 128)
  w2 = _permute_weights(Win2, bin2, Wout2, 8, 64)
  w3 = _permute_weights(Win3, bin3, Wout3, 7, 64)

  ps = _gather128(ptab, src)
  pd = _gather128(ptab, dst)
  xj1 = _gather128(x, src)
  zpad = jnp.zeros((1, 64), jnp.float32)
  b1 = jnp.concatenate([bout1.reshape(1, 64), zpad], axis=1)
  b2 = jnp.concatenate([bout2.reshape(1, 64), zpad], axis=1)
  b3 = jnp.concatenate([bout3.reshape(1, 64), zpad], axis=1)

  t1, rel16 = _dense1()(ps, pd, xj1, w1[0], w1[1], w1[2])
  pp = _scatter_k(t1, dst)
  h1 = _combine(pp, b1)

  xj2 = _gather128(h1, src)
  t2 = _dense23(8)(rel16, xj2, w2[0], w2[1], w2[2])
  pp = _scatter_k(t2, dst)
  h2 = _combine(pp, b2)

  xj3 = _gather128(h2, src)
  t3 = _dense23(7)(rel16, xj3, w3[0], w3[1], w3[2])
  pp = _scatter_k(t3, dst)

  batch3d = batch.reshape(_N // _RB, 1, _RB)
  return _pool(pp, b3, batch3d, Wfc, bfc.reshape(1, 10))


# speedup vs baseline: 1.0681x; 1.0681x over previous
import functools

import jax
import jax.numpy as jnp
from jax import lax
from jax.experimental import pallas as pl
from jax.experimental.pallas import tpu as pltpu
from jax.experimental.pallas import tpu_sc as plsc

_N = 10000
_E = 160000
_GRAPHS = 64
_NC = 2
_NS = 16
_NW = _NC * _NS
_BLK = 128
_PER_W = 5120
_E_PAD = _NW * _PER_W
_NBLK = _PER_W // _BLK
_NPAD = 10112
_RPT = _NPAD // _NS

_mesh = plsc.VectorSubcoreMesh(core_axis_name="c", subcore_axis_name="s")


def _f32(*shape):
  return jax.ShapeDtypeStruct(shape, jnp.float32)


_NBUF = 3


@functools.partial(
    pl.kernel,
    out_type=_f32(_E_PAD, 128),
    mesh=_mesh,
    scratch_types=[
        pltpu.VMEM((_NBLK, _BLK), jnp.int32),
        [pltpu.VMEM((_BLK, 128), jnp.float32) for _ in range(_NBUF)],
        pltpu.SemaphoreType.DMA,
        pltpu.SemaphoreType.DMA,
    ],
)
def _gather128(table_hbm, idx2d_hbm, out_hbm, idxv, rows, gsem, wsem):
  wid = lax.axis_index("s") * _NC + lax.axis_index("c")
  base = wid * _PER_W
  brow = wid * _NBLK
  pltpu.sync_copy(idx2d_hbm.at[pl.ds(brow, _NBLK)], idxv)

  def gather_dma(j, buf):
    return pltpu.make_async_copy(table_hbm.at[idxv.at[j]], buf, gsem)

  def write_dma(j, buf):
    return pltpu.make_async_copy(
        buf, out_hbm.at[pl.ds(base + j * _BLK, _BLK)], wsem)

  gather_dma(0, rows[0]).start()
  for j in range(_NBLK):
    b = rows[j % _NBUF]
    gather_dma(j, b).wait()
    write_dma(j, b).start()
    if j >= 2:
      write_dma(j - 2, rows[(j - 2) % _NBUF]).wait()
    if j + 1 < _NBLK:
      gather_dma(j + 1, rows[(j + 1) % _NBUF]).start()
  write_dma(_NBLK - 2, rows[(_NBLK - 2) % _NBUF]).wait()
  write_dma(_NBLK - 1, rows[(_NBLK - 1) % _NBUF]).wait()


@functools.partial(
    pl.kernel,
    out_type=_f32(2, _NPAD, 128),
    mesh=_mesh,
    scratch_types=[
        pltpu.VMEM((_NBLK, _BLK), jnp.int32),
        pltpu.VMEM((_BLK, 128), jnp.float32),
        pltpu.VMEM_SHARED((_NPAD, 128), jnp.float32),
        pltpu.SemaphoreType.DMA,
    ],
)
def _scatter_k(t_hbm, dst2d_hbm, out_hbm, idxv, tbuf, acc, sem):
  core = lax.axis_index("c")
  sid = lax.axis_index("s")
  wid = sid * _NC + core
  base = wid * _PER_W
  zero16 = jnp.zeros((16,), jnp.float32)

  def zrow(r, carry):
    for k in range(8):
      tbuf[r, pl.ds(k * 16, 16)] = zero16
    return carry

  lax.fori_loop(0, _BLK, zrow, 0)
  r0 = sid * _RPT
  brow = wid * _NBLK
  for z in range(5):
    zo = min(z * _BLK, _RPT - _BLK)
    pltpu.sync_copy(tbuf, acc.at[pl.ds(r0 + zo, _BLK)])
  pltpu.sync_copy(dst2d_hbm.at[pl.ds(brow, _NBLK)], idxv)
  plsc.subcore_barrier()

  def step(j, carry):
    off = base + j * _BLK
    pltpu.sync_copy(t_hbm.at[pl.ds(off, _BLK)], tbuf)
    pltpu.sync_copy(tbuf, acc.at[idxv.at[j]], add=True)
    return carry

  lax.fori_loop(0, _NBLK, step, 0)
  plsc.subcore_barrier()
  pltpu.sync_copy(acc.at[pl.ds(r0, _RPT)], out_hbm.at[core, pl.ds(r0, _RPT)])


@functools.partial(
    pl.kernel,
    out_type=_f32(_E_PAD, 128),
    mesh=_mesh,
    scratch_types=[
        pltpu.VMEM((_NBLK, _BLK), jnp.int32),
        pltpu.VMEM((_NBLK, _BLK), jnp.int32),
        [pltpu.VMEM((_BLK, 128), jnp.float32) for _ in range(_NBUF)],
        [pltpu.VMEM((_BLK, 128), jnp.float32) for _ in range(_NBUF)],
        pltpu.SemaphoreType.DMA,
        pltpu.SemaphoreType.DMA,
        pltpu.SemaphoreType.DMA,
    ],
)
def _relgather(table_hbm, src2_hbm, dst2_hbm, out_hbm, sidxv, didxv,
               srows, drows, gsem, dsem, wsem):
  wid = lax.axis_index("s") * _NC + lax.axis_index("c")
  base = wid * _PER_W
  brow = wid * _NBLK
  pltpu.sync_copy(src2_hbm.at[pl.ds(brow, _NBLK)], sidxv)
  pltpu.sync_copy(dst2_hbm.at[pl.ds(brow, _NBLK)], didxv)

  def gs(j, buf):
    return pltpu.make_async_copy(table_hbm.at[sidxv.at[j]], buf, gsem)

  def gd(j, buf):
    return pltpu.make_async_copy(table_hbm.at[didxv.at[j]], buf, dsem)

  def wr(j, buf):
    return pltpu.make_async_copy(
        buf, out_hbm.at[pl.ds(base + j * _BLK, _BLK)], wsem)

  gs(0, srows[0]).start()
  gd(0, drows[0]).start()
  for j in range(_NBLK):
    bs = srows[j % _NBUF]
    bd = drows[j % _NBUF]
    gs(j, bs).wait()
    gd(j, bd).wait()

    def sub(r, c):
      bs[r, pl.ds(0, 16)] = bs[r, pl.ds(0, 16)] - bd[r, pl.ds(0, 16)]
      return c

    lax.fori_loop(0, _BLK, sub, 0)
    wr(j, bs).start()
    if j >= 2:
      wr(j - 2, srows[(j - 2) % _NBUF]).wait()
    if j + 1 < _NBLK:
      gs(j + 1, srows[(j + 1) % _NBUF]).start()
      gd(j + 1, drows[(j + 1) % _NBUF]).start()
  wr(_NBLK - 2, srows[(_NBLK - 2) % _NBUF]).wait()
  wr(_NBLK - 1, srows[(_NBLK - 1) % _NBUF]).wait()


_BE = 1024


@functools.lru_cache(maxsize=None)
def _dense1():
  hc = 7 * 128

  def body(relw_ref, xj_ref, win_ref, bin_ref, wout_ref,
           t_ref, rel_ref):
    rel = relw_ref[:, :16]
    rel_ref[...] = rel
    ss = jnp.dot(rel, win_ref[...], preferred_element_type=jnp.float32)
    ss = jnp.maximum(ss + bin_ref[...], 0.0)
    xj = xj_ref[...]
    acc = jnp.zeros((_BE, 64), jnp.float32)
    for h in range(7):
      m = ss[:, h * 128:(h + 1) * 128] * xj
      acc = acc + jnp.dot(m, wout_ref[h], preferred_element_type=jnp.float32)
    t_ref[...] = jnp.concatenate(
        [acc, jnp.zeros((_BE, 64), jnp.float32)], axis=1)

  return pl.pallas_call(
      body,
      grid=(_E_PAD // _BE,),
      in_specs=[
          pl.BlockSpec((_BE, 128), lambda i: (i, 0)),
          pl.BlockSpec((_BE, 128), lambda i: (i, 0)),
          pl.BlockSpec((16, hc), lambda i: (0, 0)),
          pl.BlockSpec((1, hc), lambda i: (0, 0)),
          pl.BlockSpec((7, 128, 64), lambda i: (0, 0, 0)),
      ],
      out_specs=[
          pl.BlockSpec((_BE, 128), lambda i: (i, 0)),
          pl.BlockSpec((_BE, 16), lambda i: (i, 0)),
      ],
      out_shape=[_f32(_E_PAD, 128), _f32(_E_PAD, 16)],
      compiler_params=pltpu.CompilerParams(
          dimension_semantics=("arbitrary",)),
  )


@functools.lru_cache(maxsize=None)
def _dense23(h_cnt):
  hc = h_cnt * 64

  def body(rel_ref, xj_ref, win_ref, bin_ref, wout_ref, t_ref):
    ss = jnp.dot(rel_ref[...], win_ref[...],
                 preferred_element_type=jnp.float32)
    ss = jnp.maximum(ss + bin_ref[...], 0.0)
    xj = xj_ref[:, :64]
    acc = jnp.zeros((_BE, 64), jnp.float32)
    for h in range(h_cnt):
      m = ss[:, h * 64:(h + 1) * 64] * xj
      acc = acc + jnp.dot(m, wout_ref[h], preferred_element_type=jnp.float32)
    t_ref[...] = jnp.concatenate(
        [acc, jnp.zeros((_BE, 64), jnp.float32)], axis=1)

  return pl.pallas_call(
      body,
      grid=(_E_PAD // _BE,),
      in_specs=[
          pl.BlockSpec((_BE, 16), lambda i: (i, 0)),
          pl.BlockSpec((_BE, 128), lambda i: (i, 0)),
          pl.BlockSpec((16, hc), lambda i: (0, 0)),
          pl.BlockSpec((1, hc), lambda i: (0, 0)),
          pl.BlockSpec((h_cnt, 64, 64), lambda i: (0, 0, 0)),
      ],
      out_specs=pl.BlockSpec((_BE, 128), lambda i: (i, 0)),
      out_shape=_f32(_E_PAD, 128),
      compiler_params=pltpu.CompilerParams(
          dimension_semantics=("arbitrary",)),
  )


_RB = 2000


def _combine_body(p_ref, b_ref, h_ref):
  h_ref[...] = jnp.maximum(p_ref[0] + p_ref[1] + b_ref[...], 0.0)


_combine = pl.pallas_call(
    _combine_body,
    grid=(_N // _RB,),
    in_specs=[
        pl.BlockSpec((2, _RB, 128), lambda i: (0, i, 0)),
        pl.BlockSpec((1, 128), lambda i: (0, 0)),
    ],
    out_specs=pl.BlockSpec((_RB, 128), lambda i: (i, 0)),
    out_shape=_f32(_N, 128),
    compiler_params=pltpu.CompilerParams(dimension_semantics=("arbitrary",)),
)


def _pool_body(p_ref, b_ref, batch_ref, wfc_ref, bfc_ref,
               out_ref, acc_ref):
  i = pl.program_id(0)

  @pl.when(i == 0)
  def _():
    acc_ref[...] = jnp.zeros_like(acc_ref)

  h3 = jnp.maximum(p_ref[0] + p_ref[1] + b_ref[...], 0.0)
  hc = jnp.concatenate([h3, jnp.ones((_RB, 64), jnp.float32)], axis=1)
  ids = batch_ref[0]
  g = lax.broadcasted_iota(jnp.int32, (_GRAPHS, _RB), 0)
  m = (g == ids).astype(jnp.float32)
  acc_ref[...] += jnp.dot(m, hc, preferred_element_type=jnp.float32)

  @pl.when(i == pl.num_programs(0) - 1)
  def _():
    sums = acc_ref[:, :64]
    cnt = acc_ref[:, 128:129]
    pooled = sums / jnp.maximum(cnt, 1.0)
    logits = jnp.dot(pooled, wfc_ref[...],
                     preferred_element_type=jnp.float32) + bfc_ref[...]
    mx = jnp.max(logits, axis=1, keepdims=True)
    lse = jnp.log(jnp.sum(jnp.exp(logits - mx), axis=1, keepdims=True))
    out_ref[...] = logits - mx - lse


_pool = pl.pallas_call(
    _pool_body,
    grid=(_N // _RB,),
    in_specs=[
        pl.BlockSpec((2, _RB, 128), lambda i: (0, i, 0)),
        pl.BlockSpec((1, 128), lambda i: (0, 0)),
        pl.BlockSpec((1, 1, _RB), lambda i: (i, 0, 0)),
        pl.BlockSpec((64, 10), lambda i: (0, 0)),
        pl.BlockSpec((1, 10), lambda i: (0, 0)),
    ],
    out_specs=pl.BlockSpec((_GRAPHS, 10), lambda i: (0, 0)),
    out_shape=_f32(_GRAPHS, 10),
    scratch_shapes=[pltpu.VMEM((_GRAPHS, 192), jnp.float32)],
    compiler_params=pltpu.CompilerParams(dimension_semantics=("arbitrary",)),
)


def _permute_weights(win, bi, wout, h_cnt, c_in):
  winp = win.reshape(3, c_in, h_cnt).transpose(0, 2, 1).reshape(3, h_cnt * c_in)
  winp = jnp.concatenate(
      [winp, jnp.zeros((13, h_cnt * c_in), jnp.float32)], axis=0)
  bip = bi.reshape(c_in, h_cnt).transpose(1, 0).reshape(1, h_cnt * c_in)
  woutp = wout.reshape(c_in, h_cnt, 64).transpose(1, 0, 2)
  return winp, bip, woutp


def kernel(x, pos, edge_index, batch, Win1, bin1, Wout1, bout1, Win2, bin2,
           Wout2, bout2, Win3, bin3, Wout3, bout3, Wfc, bfc):
  src = jnp.concatenate(
      [edge_index[0], jnp.zeros((_E_PAD - _E,), jnp.int32)]).reshape(
          _E_PAD // _BLK, _BLK)
  dst = jnp.concatenate(
      [edge_index[1], jnp.full((_E_PAD - _E,), _N, jnp.int32)]).reshape(
          _E_PAD // _BLK, _BLK)
  ptab = jnp.concatenate([pos, jnp.zeros((_N, 125), jnp.float32)], axis=1)

  w1 = _permute_weights(Win1, bin1, Wout1, 7, 128)
  w2 = _permute_weights(Win2, bin2, Wout2, 8, 64)
  w3 = _permute_weights(Win3, bin3, Wout3, 7, 64)

  relw = _relgather(ptab, src, dst)
  xj1 = _gather128(x, src)
  zpad = jnp.zeros((1, 64), jnp.float32)
  b1 = jnp.concatenate([bout1.reshape(1, 64), zpad], axis=1)
  b2 = jnp.concatenate([bout2.reshape(1, 64), zpad], axis=1)
  b3 = jnp.concatenate([bout3.reshape(1, 64), zpad], axis=1)

  t1, rel16 = _dense1()(relw, xj1, w1[0], w1[1], w1[2])
  pp = _scatter_k(t1, dst)
  h1 = _combine(pp, b1)

  xj2 = _gather128(h1, src)
  t2 = _dense23(8)(rel16, xj2, w2[0], w2[1], w2[2])
  pp = _scatter_k(t2, dst)
  h2 = _combine(pp, b2)

  xj3 = _gather128(h2, src)
  t3 = _dense23(7)(rel16, xj3, w3[0], w3[1], w3[2])
  pp = _scatter_k(t3, dst)

  batch3d = batch.reshape(_N // _RB, 1, _RB)
  return _pool(pp, b3, batch3d, Wfc, bfc.reshape(1, 10))
